# baseline (speedup 1.0000x reference)
import functools
import jax
import jax.numpy as jnp
from jax import lax
from jax.experimental import pallas as pl
from jax.experimental.pallas import tpu as pltpu
from jax.experimental.pallas import tpu_sc as plsc

NC, NS, NW, LANES = 2, 16, 32, 16


def kernel(tokens, embedding):
    B, L = tokens.shape
    V, D = embedding.shape
    BPW = B // NW
    RND = 128
    ROWS_PER_RND = RND // L
    NROUNDS = (BPW * L) // RND
    NVREG = D // LANES
    W = 2 * D
    VMAIN = (V // RND) * RND
    NBLK = VMAIN // RND
    BASE_CNT = NBLK // NW
    EXTRA = NBLK - BASE_CNT * NW

    tok = tokens.astype(jnp.int32).reshape(NW, NROUNDS, RND)
    table_t = embedding.T
    tail = jnp.pad(embedding[VMAIN:, :], ((0, 0), (0, W - D))).reshape(-1)

    mesh = plsc.VectorSubcoreMesh(core_axis_name="c", subcore_axis_name="s")
    params = pltpu.CompilerParams(
        use_tc_tiling_on_sc=True, needs_layout_passes=False)

    @functools.partial(
        pl.kernel,
        out_type=jax.ShapeDtypeStruct((V * W,), jnp.float32),
        mesh=mesh,
        compiler_params=params,
        scratch_types=[
            pltpu.VMEM((D, RND), jnp.float32),
            pltpu.VMEM((RND * W,), jnp.float32),
        ],
    )
    def fmt_kernel(tab_hbm, tail_hbm, fmt_hbm, blk_v, buf_v):
        wid = lax.axis_index("s") * NC + lax.axis_index("c")
        nblk = BASE_CNT + (wid < EXTRA).astype(jnp.int32)
        iot_w = lax.iota(jnp.int32, LANES) * W

        @pl.when(wid == NW - 1)
        def _():
            pltpu.sync_copy(tail_hbm, fmt_hbm.at[pl.ds(VMAIN * W, (V - VMAIN) * W)])

        def blk_body(k, carry):
            j = wid + NW * k
            base = pl.multiple_of(j * RND, RND)
            pltpu.sync_copy(tab_hbm.at[:, pl.ds(base, RND)], blk_v)

            def g_body(g, carry2):
                goff = g * LANES * W
                for d in range(D):
                    vals = blk_v[d, pl.ds(g * LANES, LANES)]
                    plsc.store_scatter(buf_v, [iot_w + (goff + d)], vals)
                return carry2

            lax.fori_loop(0, RND // LANES, g_body, 0)
            pltpu.sync_copy(buf_v, fmt_hbm.at[pl.ds(base * W, RND * W)])
            return carry

        lax.fori_loop(0, nblk, blk_body, 0)

    @functools.partial(
        pl.kernel,
        out_type=jax.ShapeDtypeStruct((B, D), jnp.float32),
        mesh=mesh,
        compiler_params=params,
        scratch_types=[
            pltpu.VMEM((NROUNDS, RND), jnp.int32),
            pltpu.VMEM((RND, W), jnp.float32),
            pltpu.VMEM((RND, W), jnp.float32),
            pltpu.VMEM((BPW, D), jnp.float32),
            pltpu.SemaphoreType.DMA,
            pltpu.SemaphoreType.DMA,
        ],
    )
    def pool_kernel(tok_hbm, fmt_hbm, out_hbm, idx_v, rb0, rb1, out_v,
                    sem0, sem1):
        wid = lax.axis_index("s") * NC + lax.axis_index("c")
        pltpu.sync_copy(tok_hbm.at[wid], idx_v)
        inv_l = jnp.float32(1.0 / L)
        rbs = (rb0, rb1)
        sems = (sem0, sem1)

        pltpu.async_copy(fmt_hbm.at[idx_v.at[0]], rb0, sem0)

        def pair_body(h, carry):
            for b in range(2):
                j = 2 * h + b
                nxt = j + 1

                @pl.when(nxt < NROUNDS)
                def _():
                    pltpu.async_copy(
                        fmt_hbm.at[idx_v.at[nxt]], rbs[1 - b], sems[1 - b])

                pltpu.make_async_copy(
                    fmt_hbm.at[idx_v.at[j]], rbs[b], sems[b]).wait()
                rb = rbs[b]
                for r in range(ROWS_PER_RND):
                    for c in range(NVREG):
                        acc = rb[r * L, pl.ds(c * LANES, LANES)]
                        for k in range(1, L):
                            acc = acc + rb[r * L + k, pl.ds(c * LANES, LANES)]
                        out_v[j * ROWS_PER_RND + r, pl.ds(c * LANES, LANES)] = (
                            acc * inv_l)
            return carry

        lax.fori_loop(0, NROUNDS // 2, pair_body, 0)
        pltpu.sync_copy(out_v, out_hbm.at[pl.ds(wid * BPW, BPW)])

    fmt = fmt_kernel(table_t, tail)
    return pool_kernel(tok, fmt.reshape(V, W))


# speedup vs baseline: 1.0693x; 1.0693x over previous
import functools
import jax
import jax.numpy as jnp
from jax import lax
from jax.experimental import pallas as pl
from jax.experimental.pallas import tpu as pltpu
from jax.experimental.pallas import tpu_sc as plsc

NC, NS, NW, LANES = 2, 16, 32, 16


def kernel(tokens, embedding):
    B, L = tokens.shape
    V, D = embedding.shape
    BPW = B // NW
    RND = 128
    ROWS_PER_RND = RND // L
    NROUNDS = (BPW * L) // RND
    NVREG = D // LANES
    W = 2 * D
    VMAIN = (V // RND) * RND
    NBLK = VMAIN // RND
    BASE_CNT = NBLK // NW
    EXTRA = NBLK - BASE_CNT * NW

    tok = tokens.astype(jnp.int32).reshape(NW, NROUNDS, RND)
    table_t = embedding.T
    tail = jnp.pad(embedding[VMAIN:, :], ((0, 0), (0, W - D))).reshape(-1)

    mesh = plsc.VectorSubcoreMesh(core_axis_name="c", subcore_axis_name="s")
    params = pltpu.CompilerParams(
        use_tc_tiling_on_sc=True, needs_layout_passes=False)

    @functools.partial(
        pl.kernel,
        out_type=jax.ShapeDtypeStruct((V * W,), jnp.float32),
        mesh=mesh,
        compiler_params=params,
        scratch_types=[
            pltpu.VMEM((D, RND), jnp.float32),
            pltpu.VMEM((RND * W,), jnp.float32),
        ],
    )
    def fmt_kernel(tab_hbm, tail_hbm, fmt_hbm, blk_v, buf_v):
        wid = lax.axis_index("s") * NC + lax.axis_index("c")
        nblk = BASE_CNT + (wid < EXTRA).astype(jnp.int32)
        iot_w = lax.iota(jnp.int32, LANES) * W

        @pl.when(wid == NW - 1)
        def _():
            pltpu.sync_copy(tail_hbm, fmt_hbm.at[pl.ds(VMAIN * W, (V - VMAIN) * W)])

        def blk_body(k, carry):
            j = wid + NW * k
            base = pl.multiple_of(j * RND, RND)
            pltpu.sync_copy(tab_hbm.at[:, pl.ds(base, RND)], blk_v)

            @plsc.parallel_loop(0, RND // LANES, unroll=2)
            def g_body(g):
                goff = g * LANES * W
                for d in range(D):
                    vals = blk_v[d, pl.ds(g * LANES, LANES)]
                    plsc.store_scatter(buf_v, [iot_w + (goff + d)], vals)
            pltpu.sync_copy(buf_v, fmt_hbm.at[pl.ds(base * W, RND * W)])
            return carry

        lax.fori_loop(0, nblk, blk_body, 0)

    @functools.partial(
        pl.kernel,
        out_type=jax.ShapeDtypeStruct((B, D), jnp.float32),
        mesh=mesh,
        compiler_params=params,
        scratch_types=[
            pltpu.VMEM((NROUNDS, RND), jnp.int32),
            pltpu.VMEM((RND, W), jnp.float32),
            pltpu.VMEM((RND, W), jnp.float32),
            pltpu.VMEM((BPW, D), jnp.float32),
            pltpu.SemaphoreType.DMA,
            pltpu.SemaphoreType.DMA,
        ],
    )
    def pool_kernel(tok_hbm, fmt_hbm, out_hbm, idx_v, rb0, rb1, out_v,
                    sem0, sem1):
        wid = lax.axis_index("s") * NC + lax.axis_index("c")
        pltpu.sync_copy(tok_hbm.at[wid], idx_v)
        inv_l = jnp.float32(1.0 / L)
        rbs = (rb0, rb1)
        sems = (sem0, sem1)

        pltpu.async_copy(fmt_hbm.at[idx_v.at[0]], rb0, sem0)

        def pair_body(h, carry):
            for b in range(2):
                j = 2 * h + b
                nxt = j + 1

                @pl.when(nxt < NROUNDS)
                def _():
                    pltpu.async_copy(
                        fmt_hbm.at[idx_v.at[nxt]], rbs[1 - b], sems[1 - b])

                pltpu.make_async_copy(
                    fmt_hbm.at[idx_v.at[j]], rbs[b], sems[b]).wait()
                rb = rbs[b]
                for r in range(ROWS_PER_RND):
                    for c in range(NVREG):
                        acc = rb[r * L, pl.ds(c * LANES, LANES)]
                        for k in range(1, L):
                            acc = acc + rb[r * L + k, pl.ds(c * LANES, LANES)]
                        out_v[j * ROWS_PER_RND + r, pl.ds(c * LANES, LANES)] = (
                            acc * inv_l)
            return carry

        lax.fori_loop(0, NROUNDS // 2, pair_body, 0)
        pltpu.sync_copy(out_v, out_hbm.at[pl.ds(wid * BPW, BPW)])

    fmt = fmt_kernel(table_t, tail)
    return pool_kernel(tok, fmt.reshape(V, W))


# speedup vs baseline: 1.2856x; 1.2023x over previous
import functools
import jax
import jax.numpy as jnp
from jax import lax
from jax.experimental import pallas as pl
from jax.experimental.pallas import tpu as pltpu
from jax.experimental.pallas import tpu_sc as plsc

NC, NS, NW, LANES = 2, 16, 32, 16


def kernel(tokens, embedding):
    B, L = tokens.shape
    V, D = embedding.shape
    BPW = B // NW
    RND = 128
    ROWS_PER_RND = RND // L
    NROUNDS = (BPW * L) // RND
    NVREG = D // LANES
    W = 2 * D
    VMAIN = (V // RND) * RND
    NBLK = VMAIN // RND
    BASE_CNT = NBLK // NW
    EXTRA = NBLK - BASE_CNT * NW

    tok = tokens.astype(jnp.int32).reshape(NW, NROUNDS, RND)
    table_t = embedding.T
    tail = jnp.pad(embedding[VMAIN:, :], ((0, 0), (0, W - D))).reshape(-1)

    mesh = plsc.VectorSubcoreMesh(core_axis_name="c", subcore_axis_name="s")
    params = pltpu.CompilerParams(
        use_tc_tiling_on_sc=True, needs_layout_passes=False)

    @functools.partial(
        pl.kernel,
        out_type=jax.ShapeDtypeStruct((V * W,), jnp.float32),
        mesh=mesh,
        compiler_params=params,
        scratch_types=[
            pltpu.VMEM((D, RND + 2), jnp.float32),
            pltpu.VMEM((RND * W,), jnp.float32),
        ],
    )
    def fmt_kernel(tab_hbm, tail_hbm, fmt_hbm, blk_v, buf_v):
        wid = lax.axis_index("s") * NC + lax.axis_index("c")
        nblk = BASE_CNT + (wid < EXTRA).astype(jnp.int32)
        iot = lax.iota(jnp.int32, LANES)
        idxc = [c * LANES + iot for c in range(NVREG)]

        @pl.when(wid == NW - 1)
        def _():
            pltpu.sync_copy(tail_hbm, fmt_hbm.at[pl.ds(VMAIN * W, (V - VMAIN) * W)])

        def blk_body(k, carry):
            j = wid + NW * k
            base = pl.multiple_of(j * RND, RND)
            pltpu.sync_copy(tab_hbm.at[:, pl.ds(base, RND)],
                            blk_v.at[:, pl.ds(0, RND)])

            @plsc.parallel_loop(0, RND, unroll=4)
            def v_body(v):
                col = jnp.full((LANES,), v, jnp.int32)
                for c in range(NVREG):
                    vals = plsc.load_gather(blk_v, [idxc[c], col])
                    buf_v[pl.ds(v * W + c * LANES, LANES)] = vals
            pltpu.sync_copy(buf_v, fmt_hbm.at[pl.ds(base * W, RND * W)])
            return carry

        lax.fori_loop(0, nblk, blk_body, 0)

    @functools.partial(
        pl.kernel,
        out_type=jax.ShapeDtypeStruct((B, D), jnp.float32),
        mesh=mesh,
        compiler_params=params,
        scratch_types=[
            pltpu.VMEM((NROUNDS, RND), jnp.int32),
            pltpu.VMEM((RND, W), jnp.float32),
            pltpu.VMEM((RND, W), jnp.float32),
            pltpu.VMEM((BPW, D), jnp.float32),
            pltpu.SemaphoreType.DMA,
            pltpu.SemaphoreType.DMA,
        ],
    )
    def pool_kernel(tok_hbm, fmt_hbm, out_hbm, idx_v, rb0, rb1, out_v,
                    sem0, sem1):
        wid = lax.axis_index("s") * NC + lax.axis_index("c")
        pltpu.sync_copy(tok_hbm.at[wid], idx_v)
        inv_l = jnp.float32(1.0 / L)
        rbs = (rb0, rb1)
        sems = (sem0, sem1)

        pltpu.async_copy(fmt_hbm.at[idx_v.at[0]], rb0, sem0)

        def pair_body(h, carry):
            for b in range(2):
                j = 2 * h + b
                nxt = j + 1

                @pl.when(nxt < NROUNDS)
                def _():
                    pltpu.async_copy(
                        fmt_hbm.at[idx_v.at[nxt]], rbs[1 - b], sems[1 - b])

                pltpu.make_async_copy(
                    fmt_hbm.at[idx_v.at[j]], rbs[b], sems[b]).wait()
                rb = rbs[b]
                for r in range(ROWS_PER_RND):
                    for c in range(NVREG):
                        acc = rb[r * L, pl.ds(c * LANES, LANES)]
                        for k in range(1, L):
                            acc = acc + rb[r * L + k, pl.ds(c * LANES, LANES)]
                        out_v[j * ROWS_PER_RND + r, pl.ds(c * LANES, LANES)] = (
                            acc * inv_l)
            return carry

        lax.fori_loop(0, NROUNDS // 2, pair_body, 0)
        pltpu.sync_copy(out_v, out_hbm.at[pl.ds(wid * BPW, BPW)])

    fmt = fmt_kernel(table_t, tail)
    return pool_kernel(tok, fmt.reshape(V, W))


# speedup vs baseline: 1.7839x; 1.3877x over previous
import functools
import jax
import jax.numpy as jnp
from jax import lax
from jax.experimental import pallas as pl
from jax.experimental.pallas import tpu as pltpu
from jax.experimental.pallas import tpu_sc as plsc

NC, NS, NW, LANES = 2, 16, 32, 16


def kernel(tokens, embedding):
    B, L = tokens.shape
    V, D = embedding.shape
    BPW = B // NW
    RND = 128
    ROWS_PER_RND = RND // L
    NROUNDS = (BPW * L) // RND
    NVREG = D // LANES
    W = 2 * D
    VMAIN = (V // RND) * RND
    NBLK = VMAIN // RND
    BASE_CNT = NBLK // NW
    EXTRA = NBLK - BASE_CNT * NW
    BLK_W = RND * W

    tok = tokens.astype(jnp.int32).reshape(NW, NROUNDS, RND)
    table_t = embedding.T
    tail = jnp.pad(embedding[VMAIN:, :], ((0, 0), (0, W - D))).reshape(-1)

    mesh = plsc.VectorSubcoreMesh(core_axis_name="c", subcore_axis_name="s")
    params = pltpu.CompilerParams(
        use_tc_tiling_on_sc=True, needs_layout_passes=False)

    @functools.partial(
        pl.kernel,
        out_type=jax.ShapeDtypeStruct((V * W,), jnp.float32),
        mesh=mesh,
        compiler_params=params,
        scratch_types=[
            pltpu.VMEM((D, RND + 2), jnp.float32),
            pltpu.VMEM((D, RND + 2), jnp.float32),
            pltpu.VMEM((BLK_W,), jnp.float32),
            pltpu.VMEM((BLK_W,), jnp.float32),
            pltpu.SemaphoreType.DMA,
            pltpu.SemaphoreType.DMA,
            pltpu.SemaphoreType.DMA,
            pltpu.SemaphoreType.DMA,
        ],
    )
    def fmt_kernel(tab_hbm, tail_hbm, fmt_hbm, blk0, blk1, buf0, buf1,
                   si0, si1, so0, so1):
        wid = lax.axis_index("s") * NC + lax.axis_index("c")
        nblk = BASE_CNT + (wid < EXTRA).astype(jnp.int32)
        iot = lax.iota(jnp.int32, LANES)
        idxc = [c * LANES + iot for c in range(NVREG)]
        blks = (blk0, blk1)
        bufs = (buf0, buf1)
        sis = (si0, si1)
        sos = (so0, so1)

        @pl.when(wid == NW - 1)
        def _():
            pltpu.sync_copy(tail_hbm, fmt_hbm.at[pl.ds(VMAIN * W, (V - VMAIN) * W)])

        def base_of(k):
            return pl.multiple_of((wid + NW * k) * RND, RND)

        def issue_in(k, b):
            pltpu.async_copy(tab_hbm.at[:, pl.ds(base_of(k), RND)],
                             blks[b].at[:, pl.ds(0, RND)], sis[b])

        issue_in(0, 0)

        def pair_body(h, carry):
            for b in range(2):
                k = 2 * h + b

                @pl.when(k < nblk)
                def _():
                    nxt = k + 1

                    @pl.when(nxt < nblk)
                    def _():
                        issue_in(nxt, 1 - b)

                    pltpu.make_async_copy(
                        tab_hbm.at[:, pl.ds(base_of(k), RND)],
                        blks[b].at[:, pl.ds(0, RND)], sis[b]).wait()

                    @pl.when(k >= 2)
                    def _():
                        pltpu.make_async_copy(
                            bufs[b], fmt_hbm.at[pl.ds(0, BLK_W)], sos[b]).wait()

                    blk = blks[b]
                    buf = bufs[b]

                    @plsc.parallel_loop(0, RND, unroll=4)
                    def v_body(v):
                        col = jnp.full((LANES,), v, jnp.int32)
                        for c in range(NVREG):
                            vals = plsc.load_gather(blk, [idxc[c], col])
                            buf[pl.ds(v * W + c * LANES, LANES)] = vals

                    pltpu.async_copy(
                        buf, fmt_hbm.at[pl.ds(base_of(k) * W, BLK_W)], sos[b])
            return carry

        lax.fori_loop(0, (BASE_CNT + 2) // 2, pair_body, 0)

        for b in range(2):
            pltpu.make_async_copy(
                bufs[b], fmt_hbm.at[pl.ds(0, BLK_W)], sos[b]).wait()

    NRB = 4

    @functools.partial(
        pl.kernel,
        out_type=jax.ShapeDtypeStruct((B * D,), jnp.float32),
        mesh=mesh,
        compiler_params=params,
        scratch_types=[
            pltpu.VMEM((NROUNDS, RND), jnp.int32),
            pltpu.VMEM((RND, W), jnp.float32),
            pltpu.VMEM((RND, W), jnp.float32),
            pltpu.VMEM((RND, W), jnp.float32),
            pltpu.VMEM((RND, W), jnp.float32),
            pltpu.VMEM((BPW * D,), jnp.float32),
            pltpu.SemaphoreType.DMA,
            pltpu.SemaphoreType.DMA,
            pltpu.SemaphoreType.DMA,
            pltpu.SemaphoreType.DMA,
        ],
    )
    def pool_kernel(tok_hbm, fmt_hbm, out_hbm, idx_v, rb0, rb1, rb2, rb3,
                    out_v, s0, s1, s2, s3):
        wid = lax.axis_index("s") * NC + lax.axis_index("c")
        pltpu.sync_copy(tok_hbm.at[wid], idx_v)
        inv_l = jnp.float32(1.0 / L)
        rbs = (rb0, rb1, rb2, rb3)
        sems = (s0, s1, s2, s3)

        for p in range(NRB - 1):
            pltpu.async_copy(fmt_hbm.at[idx_v.at[p]], rbs[p], sems[p])

        def quad_body(h, carry):
            for b in range(NRB):
                j = NRB * h + b
                nxt = j + NRB - 1

                nb = (b + NRB - 1) % NRB

                @pl.when(nxt < NROUNDS)
                def _():
                    pltpu.async_copy(
                        fmt_hbm.at[idx_v.at[nxt]], rbs[nb], sems[nb])

                pltpu.make_async_copy(
                    fmt_hbm.at[idx_v.at[j]], rbs[b], sems[b]).wait()
                rb = rbs[b]
                for r in range(ROWS_PER_RND):
                    for c in range(NVREG):
                        acc = rb[r * L, pl.ds(c * LANES, LANES)]
                        for k in range(1, L):
                            acc = acc + rb[r * L + k, pl.ds(c * LANES, LANES)]
                        out_v[pl.ds((j * ROWS_PER_RND + r) * D + c * LANES,
                                    LANES)] = acc * inv_l
            return carry

        lax.fori_loop(0, NROUNDS // NRB, quad_body, 0)
        pltpu.sync_copy(out_v, out_hbm.at[pl.ds(wid * BPW * D, BPW * D)])

    fmt = fmt_kernel(table_t, tail)
    return pool_kernel(tok, fmt.reshape(V, W)).reshape(B, D)
